# baseline (speedup 1.0000x reference)
import functools

import jax
import jax.numpy as jnp
from jax.experimental import pallas as pl
from jax.experimental.pallas import tpu as pltpu


def _round_up(v, m):
    return (v + m - 1) // m * m


_HIST_G = 512


def _hist_kernel(n_rows, n_cols, starts_ref, s_ref, o_ref, acc_ref):
    g_sz = _HIST_G
    hi_cnt = n_cols // 128
    r_blk = max(1, 256 // hi_cnt)
    key_cnt = r_blk * hi_cnt
    iota_k = jax.lax.broadcasted_iota(jnp.int32, (key_cnt, g_sz), 0)
    iota_l = jax.lax.broadcasted_iota(jnp.int32, (128, g_sz), 0)
    lane = jax.lax.broadcasted_iota(jnp.int32, (1, g_sz), 1)

    def blk_body(rb, carry):
        r0 = rb * r_blk
        a = starts_ref[r0]
        b = starts_ref[r0 + r_blk]
        acc_ref[...] = jnp.zeros_like(acc_ref)

        def grp_body(t, carry2):
            e = s_ref[t]
            pos = t * g_sz + lane
            valid = (pos >= a) & (pos < b)
            key = (e >> 7) - r0 * hi_cnt
            lo = e & 127
            ohk = ((iota_k == key) & valid).astype(jnp.bfloat16)
            ohl = (iota_l == lo).astype(jnp.bfloat16)
            part = jax.lax.dot_general(
                ohk, ohl, (((1,), (1,)), ((), ())),
                preferred_element_type=jnp.float32)
            acc_ref[...] += part
            return carry2

        jax.lax.fori_loop(a // g_sz, (b + g_sz - 1) // g_sz, grp_body, 0)
        o_ref[pl.ds(r0, r_blk)] = (
            acc_ref[...].reshape(r_blk, hi_cnt, 128).astype(jnp.bfloat16))
        return carry

    jax.lax.fori_loop(0, n_rows // r_blk, blk_body, 0)


def _adjacency_counts(g_edge_index, n_pad):
    src, dst = g_edge_index[0], g_edge_index[1]
    e = src.shape[0]
    lin = dst * n_pad + src
    s = jnp.sort(lin)
    qs = jnp.arange(n_pad + 1, dtype=jnp.int32) * n_pad
    starts = jnp.searchsorted(s, qs, side="left").astype(jnp.int32)
    e_pad = _round_up(e, _HIST_G)
    if e_pad != e:
        s = jnp.concatenate([s, jnp.zeros((e_pad - e,), jnp.int32)])
    s3 = s.reshape(e_pad // _HIST_G, 1, _HIST_G)

    hi_cnt = n_pad // 128
    grid_spec = pltpu.PrefetchScalarGridSpec(
        num_scalar_prefetch=1,
        grid=(1,),
        in_specs=[pl.BlockSpec((e_pad // _HIST_G, 1, _HIST_G),
                               lambda i, st: (0, 0, 0))],
        out_specs=pl.BlockSpec((n_pad, hi_cnt, 128), lambda i, st: (0, 0, 0)),
        scratch_shapes=[
            pltpu.VMEM((max(1, 256 // hi_cnt) * hi_cnt, 128), jnp.float32)],
    )
    a3 = pl.pallas_call(
        functools.partial(_hist_kernel, n_pad, n_pad),
        out_shape=jax.ShapeDtypeStruct((n_pad, hi_cnt, 128), jnp.bfloat16),
        grid_spec=grid_spec,
    )(starts, s3)
    return a3.reshape(n_pad, n_pad)


def _gnn_kernel(num_convs,
                idx_ref,
                a_ref, x_ref, w_ref, b_ref,
                wl_ref, bl_ref,
                o_ref):
    a = a_ref[...]
    n = a.shape[0]
    m = o_ref.shape[0]

    ones = jnp.ones((n, 128), jnp.bfloat16)
    deg = jnp.dot(a, ones, preferred_element_type=jnp.float32)[:, :1] + 1.0
    dinv = jax.lax.rsqrt(deg)

    h = x_ref[...]
    for l in range(num_convs):
        xw = jnp.dot(h.astype(jnp.bfloat16), w_ref[l],
                     preferred_element_type=jnp.float32)
        y = xw * dinv
        agg = jnp.dot(a, y.astype(jnp.bfloat16),
                      preferred_element_type=jnp.float32) + y
        h = jnp.maximum(agg * dinv + b_ref[l], 0.0)

    hb = h.astype(jnp.bfloat16)
    hs = jnp.dot(hb, wl_ref[0], preferred_element_type=jnp.float32)
    hd = jnp.dot(hb, wl_ref[1], preferred_element_type=jnp.float32)

    rows = jax.lax.broadcasted_iota(jnp.int32, (n, m), 0)
    oh_s = (rows == idx_ref[0:1, :]).astype(jnp.float32)
    oh_d = (rows == idx_ref[1:2, :]).astype(jnp.float32)
    contract = (((0,), (0,)), ((), ()))
    gs = jax.lax.dot_general(oh_s, hs, contract,
                             preferred_element_type=jnp.float32)
    gd = jax.lax.dot_general(oh_d, hd, contract,
                             preferred_element_type=jnp.float32)

    z = gs + gd + bl_ref[...]
    o_ref[...] = 0.5 * (jnp.tanh(0.5 * z) + 1.0)


def _forward(params, x, g_edge_index, index01):
    n, d = x.shape
    convs = params["convs"]
    num_convs = len(convs)
    hdim = convs[-1][0].shape[1]
    out_dim = params["linear_w"].shape[0]
    m = index01.shape[0]

    LANE = 128
    n_pad = _round_up(n, LANE)
    p = _round_up(max(d, hdim), LANE)
    o_pad = _round_up(out_dim, LANE)
    m_pad = _round_up(m, LANE)

    a_p = _adjacency_counts(g_edge_index, n_pad)

    x_p = (jnp.zeros((n_pad, p), jnp.float32).at[:n, :d].set(x)
           .astype(jnp.bfloat16))

    w_stack = jnp.zeros((num_convs, p, p), jnp.float32)
    b_stack = jnp.zeros((num_convs, 1, p), jnp.float32)
    for l, (w, b) in enumerate(convs):
        w_stack = w_stack.at[l, :w.shape[0], :w.shape[1]].set(w)
        b_stack = b_stack.at[l, 0, :b.shape[0]].set(b)
    w_stack = w_stack.astype(jnp.bfloat16)

    wl_t = params["linear_w"].T
    wl_stack = jnp.zeros((2, p, o_pad), jnp.float32)
    wl_stack = wl_stack.at[0, :hdim, :out_dim].set(wl_t[:hdim])
    wl_stack = wl_stack.at[1, :hdim, :out_dim].set(wl_t[hdim:])
    wl_stack = wl_stack.astype(jnp.bfloat16)
    bl_p = jnp.zeros((1, o_pad), jnp.float32).at[0, :out_dim].set(
        params["linear_b"])

    src, dst = g_edge_index[0], g_edge_index[1]
    idx = jnp.zeros((8, m_pad), jnp.int32)
    idx = idx.at[0, :m].set(src[index01].astype(jnp.int32))
    idx = idx.at[1, :m].set(dst[index01].astype(jnp.int32))

    out_p = pl.pallas_call(
        functools.partial(_gnn_kernel, num_convs),
        out_shape=jax.ShapeDtypeStruct((m_pad, o_pad), jnp.float32),
    )(idx, a_p, x_p, w_stack, b_stack, wl_stack, bl_p)

    return out_p[:m, :out_dim][None]


def kernel(x, w1, b1, w2, b2, linear_w, linear_b,
           g_edge_index, lg_edge_index, index01):
    del lg_edge_index
    params = {
        "convs": [(w1, b1), (w2, b2)],
        "linear_w": linear_w,
        "linear_b": linear_b,
    }
    return _forward(params, x, g_edge_index, index01)


# speedup vs baseline: 1.0283x; 1.0283x over previous
import functools

import jax
import jax.numpy as jnp
from jax.experimental import pallas as pl
from jax.experimental.pallas import tpu as pltpu


def _round_up(v, m):
    return (v + m - 1) // m * m


_HIST_G = 512


def _hist_kernel(n_rows, n_cols, starts_ref, s_ref, o_ref, acc_ref):
    g_sz = _HIST_G
    hi_cnt = n_cols // 128
    iota_h = jax.lax.broadcasted_iota(jnp.int32, (hi_cnt, g_sz), 0)
    iota_l = jax.lax.broadcasted_iota(jnp.int32, (128, g_sz), 0)
    lane = jax.lax.broadcasted_iota(jnp.int32, (1, g_sz), 1)

    def row_body(r, carry):
        a = starts_ref[r]
        b = starts_ref[r + 1]

        def grp_body(t, acc):
            e = s_ref[t]
            pos = t * g_sz + lane
            valid = (pos >= a) & (pos < b)
            hi = (e >> 7) - r * hi_cnt
            lo = e & 127
            ohh = ((iota_h == hi) & valid).astype(jnp.bfloat16)
            ohl = (iota_l == lo).astype(jnp.bfloat16)
            part = jax.lax.dot_general(
                ohh, ohl, (((1,), (1,)), ((), ())),
                preferred_element_type=jnp.float32)
            return acc + part

        acc0 = jnp.zeros((hi_cnt, 128), jnp.float32)
        acc = jax.lax.fori_loop(a // g_sz, (b + g_sz - 1) // g_sz,
                                grp_body, acc0)
        o_ref[pl.ds(r, 1)] = acc.astype(jnp.bfloat16)[None]
        return carry

    jax.lax.fori_loop(0, n_rows, row_body, 0)


def _adjacency_counts(g_edge_index, n_pad):
    src, dst = g_edge_index[0], g_edge_index[1]
    e = src.shape[0]
    lin = dst * n_pad + src
    s = jnp.sort(lin)
    qs = jnp.arange(n_pad + 1, dtype=jnp.int32) * n_pad
    starts = jnp.searchsorted(s, qs, side="left").astype(jnp.int32)
    e_pad = _round_up(e, _HIST_G)
    if e_pad != e:
        s = jnp.concatenate([s, jnp.zeros((e_pad - e,), jnp.int32)])
    s3 = s.reshape(e_pad // _HIST_G, 1, _HIST_G)

    hi_cnt = n_pad // 128
    grid_spec = pltpu.PrefetchScalarGridSpec(
        num_scalar_prefetch=1,
        grid=(1,),
        in_specs=[pl.BlockSpec((e_pad // _HIST_G, 1, _HIST_G),
                               lambda i, st: (0, 0, 0))],
        out_specs=pl.BlockSpec((n_pad, hi_cnt, 128), lambda i, st: (0, 0, 0)),
        scratch_shapes=[
            pltpu.VMEM((max(1, 256 // hi_cnt) * hi_cnt, 128), jnp.float32)],
    )
    a3 = pl.pallas_call(
        functools.partial(_hist_kernel, n_pad, n_pad),
        out_shape=jax.ShapeDtypeStruct((n_pad, hi_cnt, 128), jnp.bfloat16),
        grid_spec=grid_spec,
    )(starts, s3)
    return a3.reshape(n_pad, n_pad)


def _gnn_kernel(num_convs,
                idx_ref,
                a_ref, x_ref, w_ref, b_ref,
                wl_ref, bl_ref,
                o_ref):
    a = a_ref[...]
    n = a.shape[0]
    m = o_ref.shape[0]

    ones = jnp.ones((n, 128), jnp.bfloat16)
    deg = jnp.dot(a, ones, preferred_element_type=jnp.float32)[:, :1] + 1.0
    dinv = jax.lax.rsqrt(deg)

    h = x_ref[...]
    for l in range(num_convs):
        xw = jnp.dot(h.astype(jnp.bfloat16), w_ref[l],
                     preferred_element_type=jnp.float32)
        y = xw * dinv
        agg = jnp.dot(a, y.astype(jnp.bfloat16),
                      preferred_element_type=jnp.float32) + y
        h = jnp.maximum(agg * dinv + b_ref[l], 0.0)

    hb = h.astype(jnp.bfloat16)
    hs = jnp.dot(hb, wl_ref[0], preferred_element_type=jnp.float32)
    hd = jnp.dot(hb, wl_ref[1], preferred_element_type=jnp.float32)

    rows = jax.lax.broadcasted_iota(jnp.int32, (n, m), 0)
    oh_s = (rows == idx_ref[0:1, :]).astype(jnp.float32)
    oh_d = (rows == idx_ref[1:2, :]).astype(jnp.float32)
    contract = (((0,), (0,)), ((), ()))
    gs = jax.lax.dot_general(oh_s, hs, contract,
                             preferred_element_type=jnp.float32)
    gd = jax.lax.dot_general(oh_d, hd, contract,
                             preferred_element_type=jnp.float32)

    z = gs + gd + bl_ref[...]
    o_ref[...] = 0.5 * (jnp.tanh(0.5 * z) + 1.0)


def _forward(params, x, g_edge_index, index01):
    n, d = x.shape
    convs = params["convs"]
    num_convs = len(convs)
    hdim = convs[-1][0].shape[1]
    out_dim = params["linear_w"].shape[0]
    m = index01.shape[0]

    LANE = 128
    n_pad = _round_up(n, LANE)
    p = _round_up(max(d, hdim), LANE)
    o_pad = _round_up(out_dim, LANE)
    m_pad = _round_up(m, LANE)

    a_p = _adjacency_counts(g_edge_index, n_pad)

    x_p = (jnp.zeros((n_pad, p), jnp.float32).at[:n, :d].set(x)
           .astype(jnp.bfloat16))

    w_stack = jnp.zeros((num_convs, p, p), jnp.float32)
    b_stack = jnp.zeros((num_convs, 1, p), jnp.float32)
    for l, (w, b) in enumerate(convs):
        w_stack = w_stack.at[l, :w.shape[0], :w.shape[1]].set(w)
        b_stack = b_stack.at[l, 0, :b.shape[0]].set(b)
    w_stack = w_stack.astype(jnp.bfloat16)

    wl_t = params["linear_w"].T
    wl_stack = jnp.zeros((2, p, o_pad), jnp.float32)
    wl_stack = wl_stack.at[0, :hdim, :out_dim].set(wl_t[:hdim])
    wl_stack = wl_stack.at[1, :hdim, :out_dim].set(wl_t[hdim:])
    wl_stack = wl_stack.astype(jnp.bfloat16)
    bl_p = jnp.zeros((1, o_pad), jnp.float32).at[0, :out_dim].set(
        params["linear_b"])

    src, dst = g_edge_index[0], g_edge_index[1]
    idx = jnp.zeros((8, m_pad), jnp.int32)
    idx = idx.at[0, :m].set(src[index01].astype(jnp.int32))
    idx = idx.at[1, :m].set(dst[index01].astype(jnp.int32))

    out_p = pl.pallas_call(
        functools.partial(_gnn_kernel, num_convs),
        out_shape=jax.ShapeDtypeStruct((m_pad, o_pad), jnp.float32),
    )(idx, a_p, x_p, w_stack, b_stack, wl_stack, bl_p)

    return out_p[:m, :out_dim][None]


def kernel(x, w1, b1, w2, b2, linear_w, linear_b,
           g_edge_index, lg_edge_index, index01):
    del lg_edge_index
    params = {
        "convs": [(w1, b1), (w2, b2)],
        "linear_w": linear_w,
        "linear_b": linear_b,
    }
    return _forward(params, x, g_edge_index, index01)


# speedup vs baseline: 1.0770x; 1.0473x over previous
import functools

import jax
import jax.numpy as jnp
from jax.experimental import pallas as pl
from jax.experimental.pallas import tpu as pltpu


def _round_up(v, m):
    return (v + m - 1) // m * m


_HIST_G = 1024


def _hist_kernel(n_rows, n_cols, starts_ref, s_ref, o_ref, acc_ref):
    g_sz = _HIST_G
    hi_cnt = n_cols // 128
    iota_h = jax.lax.broadcasted_iota(jnp.int32, (hi_cnt, g_sz), 0)
    iota_l = jax.lax.broadcasted_iota(jnp.int32, (128, g_sz), 0)
    lane = jax.lax.broadcasted_iota(jnp.int32, (1, g_sz), 1)

    def row_body(r, carry):
        a = starts_ref[r]
        b = starts_ref[r + 1]

        def grp_body(t, acc):
            e = s_ref[t]
            pos = t * g_sz + lane
            valid = (pos >= a) & (pos < b)
            hi = (e >> 7) - r * hi_cnt
            lo = e & 127
            ohh = ((iota_h == hi) & valid).astype(jnp.bfloat16)
            ohl = (iota_l == lo).astype(jnp.bfloat16)
            part = jax.lax.dot_general(
                ohh, ohl, (((1,), (1,)), ((), ())),
                preferred_element_type=jnp.float32)
            return acc + part

        acc0 = jnp.zeros((hi_cnt, 128), jnp.float32)
        acc = jax.lax.fori_loop(a // g_sz, (b + g_sz - 1) // g_sz,
                                grp_body, acc0)
        o_ref[pl.ds(r, 1)] = acc.astype(jnp.bfloat16)[None]
        return carry

    jax.lax.fori_loop(0, n_rows, row_body, 0)


def _adjacency_counts(g_edge_index, n_pad):
    src, dst = g_edge_index[0], g_edge_index[1]
    e = src.shape[0]
    lin = dst * n_pad + src
    s = jnp.sort(lin)
    qs = jnp.arange(n_pad + 1, dtype=jnp.int32) * n_pad
    starts = jnp.searchsorted(s, qs, side="left").astype(jnp.int32)
    e_pad = _round_up(e, _HIST_G)
    if e_pad != e:
        s = jnp.concatenate([s, jnp.zeros((e_pad - e,), jnp.int32)])
    s3 = s.reshape(e_pad // _HIST_G, 1, _HIST_G)

    hi_cnt = n_pad // 128
    grid_spec = pltpu.PrefetchScalarGridSpec(
        num_scalar_prefetch=1,
        grid=(1,),
        in_specs=[pl.BlockSpec((e_pad // _HIST_G, 1, _HIST_G),
                               lambda i, st: (0, 0, 0))],
        out_specs=pl.BlockSpec((n_pad, hi_cnt, 128), lambda i, st: (0, 0, 0)),
        scratch_shapes=[
            pltpu.VMEM((max(1, 256 // hi_cnt) * hi_cnt, 128), jnp.float32)],
    )
    a3 = pl.pallas_call(
        functools.partial(_hist_kernel, n_pad, n_pad),
        out_shape=jax.ShapeDtypeStruct((n_pad, hi_cnt, 128), jnp.bfloat16),
        grid_spec=grid_spec,
    )(starts, s3)
    return a3.reshape(n_pad, n_pad)


def _gnn_kernel(num_convs,
                idx_ref,
                a_ref, x_ref, w_ref, b_ref,
                wl_ref, bl_ref,
                o_ref):
    a = a_ref[...]
    n = a.shape[0]
    m = o_ref.shape[0]

    ones = jnp.ones((n, 128), jnp.bfloat16)
    deg = jnp.dot(a, ones, preferred_element_type=jnp.float32)[:, :1] + 1.0
    dinv = jax.lax.rsqrt(deg)

    h = x_ref[...]
    for l in range(num_convs):
        xw = jnp.dot(h.astype(jnp.bfloat16), w_ref[l],
                     preferred_element_type=jnp.float32)
        y = xw * dinv
        agg = jnp.dot(a, y.astype(jnp.bfloat16),
                      preferred_element_type=jnp.float32) + y
        h = jnp.maximum(agg * dinv + b_ref[l], 0.0)

    hb = h.astype(jnp.bfloat16)
    hs = jnp.dot(hb, wl_ref[0], preferred_element_type=jnp.float32)
    hd = jnp.dot(hb, wl_ref[1], preferred_element_type=jnp.float32)

    rows = jax.lax.broadcasted_iota(jnp.int32, (n, m), 0)
    oh_s = (rows == idx_ref[0:1, :]).astype(jnp.float32)
    oh_d = (rows == idx_ref[1:2, :]).astype(jnp.float32)
    contract = (((0,), (0,)), ((), ()))
    gs = jax.lax.dot_general(oh_s, hs, contract,
                             preferred_element_type=jnp.float32)
    gd = jax.lax.dot_general(oh_d, hd, contract,
                             preferred_element_type=jnp.float32)

    z = gs + gd + bl_ref[...]
    o_ref[...] = 0.5 * (jnp.tanh(0.5 * z) + 1.0)


def _forward(params, x, g_edge_index, index01):
    n, d = x.shape
    convs = params["convs"]
    num_convs = len(convs)
    hdim = convs[-1][0].shape[1]
    out_dim = params["linear_w"].shape[0]
    m = index01.shape[0]

    LANE = 128
    n_pad = _round_up(n, LANE)
    p = _round_up(max(d, hdim), LANE)
    o_pad = _round_up(out_dim, LANE)
    m_pad = _round_up(m, LANE)

    a_p = _adjacency_counts(g_edge_index, n_pad)

    x_p = (jnp.zeros((n_pad, p), jnp.float32).at[:n, :d].set(x)
           .astype(jnp.bfloat16))

    w_stack = jnp.zeros((num_convs, p, p), jnp.float32)
    b_stack = jnp.zeros((num_convs, 1, p), jnp.float32)
    for l, (w, b) in enumerate(convs):
        w_stack = w_stack.at[l, :w.shape[0], :w.shape[1]].set(w)
        b_stack = b_stack.at[l, 0, :b.shape[0]].set(b)
    w_stack = w_stack.astype(jnp.bfloat16)

    wl_t = params["linear_w"].T
    wl_stack = jnp.zeros((2, p, o_pad), jnp.float32)
    wl_stack = wl_stack.at[0, :hdim, :out_dim].set(wl_t[:hdim])
    wl_stack = wl_stack.at[1, :hdim, :out_dim].set(wl_t[hdim:])
    wl_stack = wl_stack.astype(jnp.bfloat16)
    bl_p = jnp.zeros((1, o_pad), jnp.float32).at[0, :out_dim].set(
        params["linear_b"])

    src, dst = g_edge_index[0], g_edge_index[1]
    idx = jnp.zeros((8, m_pad), jnp.int32)
    idx = idx.at[0, :m].set(src[index01].astype(jnp.int32))
    idx = idx.at[1, :m].set(dst[index01].astype(jnp.int32))

    out_p = pl.pallas_call(
        functools.partial(_gnn_kernel, num_convs),
        out_shape=jax.ShapeDtypeStruct((m_pad, o_pad), jnp.float32),
    )(idx, a_p, x_p, w_stack, b_stack, wl_stack, bl_p)

    return out_p[:m, :out_dim][None]


def kernel(x, w1, b1, w2, b2, linear_w, linear_b,
           g_edge_index, lg_edge_index, index01):
    del lg_edge_index
    params = {
        "convs": [(w1, b1), (w2, b2)],
        "linear_w": linear_w,
        "linear_b": linear_b,
    }
    return _forward(params, x, g_edge_index, index01)
